# Initial kernel scaffold; baseline (speedup 1.0000x reference)
import jax
import jax.numpy as jnp
from jax.experimental import pallas as pl


def kernel(x_cat, tables):
    raise NotImplementedError("write your pallas kernel here")


# speedup vs baseline: 1.0265x; 1.0265x over previous
import jax
import jax.numpy as jnp
from jax import lax
from jax.experimental import pallas as pl
from jax.experimental.pallas import tpu as pltpu
from jax.experimental.pallas import tpu_sc as plsc

N_FIELDS = 26
VOCAB = 100000
D_MODEL = 64
BATCH = 16384

ROWS = BATCH * N_FIELDS
NC, NS, L = 2, 16, 16
NW = NC * NS
RPW = ROWS // NW
IDX_W = 128
JROWS = RPW // IDX_W
CHUNK = 512
GPC = CHUNK // IDX_W
NCHUNK = RPW // CHUNK

_mesh = plsc.VectorSubcoreMesh(core_axis_name="c", subcore_axis_name="s")


def _body(x_hbm, tab_hbm, out_hbm, idx_v, rows_v, gsem):
    wid = lax.axis_index("s") * NC + lax.axis_index("c")
    base = wid * RPW

    pltpu.sync_copy(x_hbm.at[pl.ds(wid * JROWS, JROWS)], idx_v)

    lanes = lax.iota(jnp.int32, L)

    def xform(j, _):
        row = idx_v.at[j]
        for k in range(IDX_W // L):
            r0 = base + j * IDX_W + k * L
            f = (r0 + lanes) % N_FIELDS
            row[pl.ds(k * L, L)] = row[pl.ds(k * L, L)] + f * VOCAB
        return 0

    lax.fori_loop(0, JROWS, xform, 0)

    def chunk(c, _):
        descs = []
        for q in range(GPC):
            descs.append(pltpu.async_copy(
                tab_hbm.at[idx_v.at[c * GPC + q]],
                rows_v.at[pl.ds(q * IDX_W, IDX_W)],
                gsem))
        for d in descs:
            d.wait()
        pltpu.sync_copy(rows_v, out_hbm.at[pl.ds(base + c * CHUNK, CHUNK)])
        return 0

    lax.fori_loop(0, NCHUNK, chunk, 0)


_call = pl.kernel(
    _body,
    out_type=jax.ShapeDtypeStruct((ROWS, D_MODEL), jnp.float32),
    mesh=_mesh,
    scratch_types=[
        pltpu.VMEM((JROWS, IDX_W), jnp.int32),
        pltpu.VMEM((CHUNK, D_MODEL), jnp.float32),
        pltpu.SemaphoreType.DMA,
    ],
    compiler_params=pltpu.CompilerParams(use_tc_tiling_on_sc=False),
)


@jax.jit
def kernel(x_cat, tables):
    x_flat = x_cat.reshape(ROWS // IDX_W, IDX_W)
    tab = tables.reshape(N_FIELDS * VOCAB, D_MODEL)
    out = _call(x_flat, tab)
    return out.reshape(BATCH, N_FIELDS, D_MODEL)


# speedup vs baseline: 1.0362x; 1.0095x over previous
import jax
import jax.numpy as jnp
from jax import lax
from jax.experimental import pallas as pl
from jax.experimental.pallas import tpu as pltpu
from jax.experimental.pallas import tpu_sc as plsc

N_FIELDS = 26
VOCAB = 100000
D_MODEL = 64
BATCH = 16384

ROWS = BATCH * N_FIELDS
NC, NS, L = 2, 16, 16
NW = NC * NS
RPW = ROWS // NW
IDX_W = 128
JROWS = RPW // IDX_W
CHUNK = 512
GPC = CHUNK // IDX_W
NCHUNK = RPW // CHUNK

_mesh = plsc.VectorSubcoreMesh(core_axis_name="c", subcore_axis_name="s")


def _body(x_hbm, tab_hbm, out_hbm, idx_v, rows0, rows1, gs0, gs1, ws0, ws1):
    wid = lax.axis_index("s") * NC + lax.axis_index("c")
    base = wid * RPW

    pltpu.sync_copy(x_hbm.at[pl.ds(wid * JROWS, JROWS)], idx_v)

    lanes = lax.iota(jnp.int32, L)

    def xform(j, _):
        row = idx_v.at[j]
        for k in range(IDX_W // L):
            r0 = base + j * IDX_W + k * L
            f = (r0 + lanes) % N_FIELDS
            row[pl.ds(k * L, L)] = row[pl.ds(k * L, L)] + f * VOCAB
        return 0

    lax.fori_loop(0, JROWS, xform, 0)

    bufs = (rows0, rows1)
    gsems = (gs0, gs1)
    wsems = (ws0, ws1)

    def fire(g, b):
        for q in range(GPC):
            pltpu.async_copy(
                tab_hbm.at[idx_v.at[g * GPC + q]],
                bufs[b].at[pl.ds(q * IDX_W, IDX_W)],
                gsems[b])

    def wait_full(b, sem):
        pltpu.make_async_copy(out_hbm.at[pl.ds(0, CHUNK)], bufs[b], sem).wait()

    fire(0, 0)

    def pair(p, _):
        g0 = 2 * p

        @pl.when(p >= 1)
        def _():
            wait_full(1, wsems[1])
        fire(g0 + 1, 1)
        wait_full(0, gsems[0])
        pltpu.async_copy(rows0, out_hbm.at[pl.ds(base + g0 * CHUNK, CHUNK)],
                         wsems[0])

        wait_full(0, wsems[0])

        @pl.when(p < NCHUNK // 2 - 1)
        def _():
            fire(g0 + 2, 0)
        wait_full(1, gsems[1])
        pltpu.async_copy(rows1, out_hbm.at[pl.ds(base + (g0 + 1) * CHUNK, CHUNK)],
                         wsems[1])
        return 0

    lax.fori_loop(0, NCHUNK // 2, pair, 0)
    wait_full(1, wsems[1])


_call = pl.kernel(
    _body,
    out_type=jax.ShapeDtypeStruct((ROWS, D_MODEL), jnp.float32),
    mesh=_mesh,
    scratch_types=[
        pltpu.VMEM((JROWS, IDX_W), jnp.int32),
        pltpu.VMEM((CHUNK, D_MODEL), jnp.float32),
        pltpu.VMEM((CHUNK, D_MODEL), jnp.float32),
        pltpu.SemaphoreType.DMA,
        pltpu.SemaphoreType.DMA,
        pltpu.SemaphoreType.DMA,
        pltpu.SemaphoreType.DMA,
    ],
    compiler_params=pltpu.CompilerParams(use_tc_tiling_on_sc=False),
)


@jax.jit
def kernel(x_cat, tables):
    x_flat = x_cat.reshape(ROWS // IDX_W, IDX_W)
    tab = tables.reshape(N_FIELDS * VOCAB, D_MODEL)
    out = _call(x_flat, tab)
    return out.reshape(BATCH, N_FIELDS, D_MODEL)


# speedup vs baseline: 1.0557x; 1.0188x over previous
import jax
import jax.numpy as jnp
from jax import lax
from jax.experimental import pallas as pl
from jax.experimental.pallas import tpu as pltpu
from jax.experimental.pallas import tpu_sc as plsc

N_FIELDS = 26
VOCAB = 100000
D_MODEL = 64
BATCH = 16384

NC, NS, L = 2, 16, 16
NW = NC * NS
CHUNK = BATCH // NW
IDX_W = 128
GPC = CHUNK // IDX_W

_mesh = plsc.VectorSubcoreMesh(core_axis_name="c", subcore_axis_name="s")


def _body(xt_hbm, tab_hbm, out_hbm, idx0, idx1, rows0, rows1,
          gs0, gs1, ws0, ws1):
    wid = lax.axis_index("s") * NC + lax.axis_index("c")
    b0 = wid * CHUNK

    idxs = (idx0, idx1)
    bufs = (rows0, rows1)
    gsems = (gs0, gs1)
    wsems = (ws0, ws1)

    def load_idx(f, b):
        pltpu.sync_copy(xt_hbm.at[f, pl.ds(wid * GPC, GPC)], idxs[b])
        off = f * VOCAB
        for j in range(GPC):
            row = idxs[b].at[j]
            for k in range(IDX_W // L):
                row[pl.ds(k * L, L)] = row[pl.ds(k * L, L)] + off

    def fire(b):
        for q in range(GPC):
            pltpu.async_copy(
                tab_hbm.at[idxs[b].at[q]],
                bufs[b].at[pl.ds(q * IDX_W, IDX_W)],
                gsems[b])

    def wait_full(b, sem):
        pltpu.make_async_copy(out_hbm.at[0, pl.ds(0, CHUNK)], bufs[b], sem).wait()

    def put(f, b):
        pltpu.async_copy(bufs[b], out_hbm.at[f, pl.ds(b0, CHUNK)], wsems[b])

    load_idx(0, 0)
    fire(0)

    def pair(p, _):
        f0 = 2 * p

        @pl.when(p >= 1)
        def _():
            wait_full(1, wsems[1])
        load_idx(f0 + 1, 1)
        fire(1)

        wait_full(0, gsems[0])
        put(f0, 0)
        wait_full(0, wsems[0])

        @pl.when(p < N_FIELDS // 2 - 1)
        def _():
            load_idx(f0 + 2, 0)
            fire(0)

        wait_full(1, gsems[1])
        put(f0 + 1, 1)
        return 0

    lax.fori_loop(0, N_FIELDS // 2, pair, 0)
    wait_full(1, wsems[1])


_call = pl.kernel(
    _body,
    out_type=jax.ShapeDtypeStruct((N_FIELDS, BATCH, D_MODEL), jnp.float32),
    mesh=_mesh,
    scratch_types=[
        pltpu.VMEM((GPC, IDX_W), jnp.int32),
        pltpu.VMEM((GPC, IDX_W), jnp.int32),
        pltpu.VMEM((CHUNK, D_MODEL), jnp.float32),
        pltpu.VMEM((CHUNK, D_MODEL), jnp.float32),
        pltpu.SemaphoreType.DMA,
        pltpu.SemaphoreType.DMA,
        pltpu.SemaphoreType.DMA,
        pltpu.SemaphoreType.DMA,
    ],
    compiler_params=pltpu.CompilerParams(use_tc_tiling_on_sc=False),
)


@jax.jit
def kernel(x_cat, tables):
    xt = x_cat.T.reshape(N_FIELDS, BATCH // IDX_W, IDX_W)
    tab = tables.reshape(N_FIELDS * VOCAB, D_MODEL)
    out = _call(xt, tab)
    return out.transpose(1, 0, 2)
